# baseline (speedup 1.0000x reference)
import math

import jax
import jax.numpy as jnp
from jax.experimental import pallas as pl

N_FEAT = 256
ROW_BLOCK = 4096


def _stage_matrix(pa, pb, theta):
    c = jnp.cos(theta)
    s = jnp.sin(theta)
    j = jnp.arange(N_FEAT, dtype=jnp.int32)
    a = (pa[:, None] == j[None, :]).astype(jnp.float32)
    b = (pb[:, None] == j[None, :]).astype(jnp.float32)
    top = c[:, None] * a - s[:, None] * b
    bot = s[:, None] * a + c[:, None] * b
    hp = jax.lax.Precision.HIGHEST
    return jnp.dot(a.T, top, precision=hp) + jnp.dot(b.T, bot, precision=hp)


def _combined_matrix(angles, indices_in, idx_out):
    n_in = angles.shape[0] // 2
    theta_in = jnp.sum(angles[:n_in], axis=0)
    theta_out = jnp.sum(angles[n_in:], axis=0)
    m_in = _stage_matrix(indices_in[0::2], indices_in[1::2], theta_in)
    m_out = _stage_matrix(idx_out[0::2], idx_out[1::2], theta_out)
    return jnp.dot(m_in, m_out, precision=jax.lax.Precision.HIGHEST)


def _rotate_kernel(x_ref, w_ref, o_ref):
    o_ref[...] = jnp.dot(
        x_ref[...],
        w_ref[...],
        preferred_element_type=jnp.float32,
        precision=jax.lax.Precision.DEFAULT,
    )


def kernel(data, angles, indices_in, idx_out):
    w = _combined_matrix(angles, indices_in, idx_out)
    n_rows = data.shape[0]
    grid = (n_rows // ROW_BLOCK,)
    return pl.pallas_call(
        _rotate_kernel,
        grid=grid,
        in_specs=[
            pl.BlockSpec((ROW_BLOCK, N_FEAT), lambda i: (i, 0)),
            pl.BlockSpec((N_FEAT, N_FEAT), lambda i: (0, 0)),
        ],
        out_specs=pl.BlockSpec((ROW_BLOCK, N_FEAT), lambda i: (i, 0)),
        out_shape=jax.ShapeDtypeStruct((n_rows, N_FEAT), jnp.float32),
    )(data, w)
